# baseline (speedup 1.0000x reference)
import functools

import jax
import jax.numpy as jnp
from jax import lax
from jax.experimental import pallas as pl
from jax.experimental.pallas import tpu as pltpu
from jax.experimental.pallas import tpu_sc as plsc

VOCAB = 1000000
EMBED_DIM = 64
BATCH = 4096
FIELDS = 26

_B = BATCH * FIELDS
_VT = VOCAB // 8

_info = plsc.get_sparse_core_info()
_NC, _NS = _info.num_cores, _info.num_subcores
_NW = _NC * _NS
_B_PER_W = _B // _NW
_BPW = BATCH // _NW
_BC = 16
_NCH = _BPW // _BC
_GROUPS = ((0, 7), (7, 7), (14, 7), (21, 5))
_NFMAX = 7
_L = 16


def _make_kernel():
    mesh = plsc.VectorSubcoreMesh(core_axis_name="c", subcore_axis_name="s")

    @functools.partial(
        pl.kernel,
        mesh=mesh,
        out_type=jax.ShapeDtypeStruct((FIELDS, EMBED_DIM, BATCH),
                                      jnp.float32),
        compiler_params=pltpu.CompilerParams(needs_layout_passes=False),
        scratch_types=[
            pltpu.VMEM((_B_PER_W + _L,), jnp.int32),
            pltpu.VMEM((_BC * _NFMAX, 1, EMBED_DIM), jnp.float32),
            pltpu.VMEM((_BC * _NFMAX, 1, EMBED_DIM), jnp.float32),
            pltpu.VMEM((_NFMAX, EMBED_DIM, _BPW), jnp.float32),
            pltpu.SemaphoreType.DMA,
            pltpu.SemaphoreType.DMA,
            pltpu.SemaphoreType.DMA,
        ],
    )
    def gather_kernel(table_hbm, idx_hbm, out_hbm, idx_v,
                      rb0, rb1, tbuf, g0, g1, osem):
        wid = lax.axis_index("s") * _NC + lax.axis_index("c")
        base = pl.multiple_of(wid * _B_PER_W, _B_PER_W)
        b_base = pl.multiple_of(wid * _BPW, _BPW)
        pltpu.sync_copy(idx_hbm.at[pl.ds(base, _B_PER_W)],
                        idx_v.at[pl.ds(0, _B_PER_W)])

        rbufs = (rb0, rb1)
        gsems = (g0, g1)
        iota16 = lax.iota(jnp.int32, _L)
        zv = iota16 * 0

        def fill(f0, nf, bc):
            rbuf = rbufs[bc % 2]
            sem = gsems[bc % 2]

            def body(br, carry):
                v = idx_v[pl.ds((bc * _BC + br) * FIELDS + f0, _L)]
                for df in range(nf):
                    i = v[df]
                    pltpu.async_copy(
                        table_hbm.at[pl.ds(i >> 3, 1), pl.ds(i & 7, 1),
                                     pl.ds(0, EMBED_DIM)],
                        rbuf.at[pl.ds(br * nf + df, 1)],
                        sem,
                    )
                return carry

            lax.fori_loop(0, _BC, body, 0)

        def drain_fill(nf, bc):
            pltpu.make_async_copy(
                table_hbm.at[pl.ds(0, _BC * nf), pl.ds(0, 1),
                             pl.ds(0, EMBED_DIM)],
                rbufs[bc % 2].at[pl.ds(0, _BC * nf)], gsems[bc % 2]
            ).wait()

        def transpose(nf, bc):
            rbuf = rbufs[bc % 2]
            for df in range(nf):
                r_idx = iota16 * nf + df

                def kbody(k, carry):
                    tbuf[df, k, pl.ds(bc * _BC, _BC)] = plsc.load_gather(
                        rbuf, [r_idx, zv, zv + k])
                    return carry

                lax.fori_loop(0, EMBED_DIM, kbody, 0)

        def put(f0, nf):
            return pltpu.async_copy(
                tbuf.at[pl.ds(0, nf)],
                out_hbm.at[pl.ds(f0, nf), pl.ds(0, EMBED_DIM),
                           pl.ds(b_base, _BPW)],
                osem,
            )

        for p, (f0, nf) in enumerate(_GROUPS):
            if p > 0:
                pltpu.make_async_copy(
                    out_hbm.at[pl.ds(0, _GROUPS[p - 1][1]),
                               pl.ds(0, EMBED_DIM), pl.ds(0, _BPW)],
                    tbuf.at[pl.ds(0, _GROUPS[p - 1][1])], osem
                ).wait()
            fill(f0, nf, 0)
            for bc in range(_NCH):
                drain_fill(nf, bc)
                if bc + 1 < _NCH:
                    fill(f0, nf, bc + 1)
                transpose(nf, bc)
            put(f0, nf)
        pltpu.make_async_copy(
            out_hbm.at[pl.ds(0, _GROUPS[-1][1]), pl.ds(0, EMBED_DIM),
                       pl.ds(0, _BPW)],
            tbuf.at[pl.ds(0, _GROUPS[-1][1])], osem
        ).wait()

    return gather_kernel


_gather = _make_kernel()


def kernel(inp, weight):
    idx = inp.reshape(-1).astype(jnp.int32)
    table_tiles = weight.reshape(_VT, 8, EMBED_DIM)
    outt = _gather(table_tiles, idx)
    return outt.transpose(2, 0, 1)


# speedup vs baseline: 1.0064x; 1.0064x over previous
import functools

import jax
import jax.numpy as jnp
from jax import lax
from jax.experimental import pallas as pl
from jax.experimental.pallas import tpu as pltpu
from jax.experimental.pallas import tpu_sc as plsc

VOCAB = 1000000
EMBED_DIM = 64
BATCH = 4096
FIELDS = 26

_B = BATCH * FIELDS
_VT = VOCAB // 8

_info = plsc.get_sparse_core_info()
_NC, _NS = _info.num_cores, _info.num_subcores
_NW = _NC * _NS
_B_PER_W = _B // _NW
_BPW = BATCH // _NW
_BC = 16
_NCH = _BPW // _BC
_GROUPS = ((0, 7), (7, 7), (14, 7), (21, 5))
_NFMAX = 7
_L = 16


def _make_kernel():
    mesh = plsc.VectorSubcoreMesh(core_axis_name="c", subcore_axis_name="s")

    @functools.partial(
        pl.kernel,
        mesh=mesh,
        out_type=jax.ShapeDtypeStruct((FIELDS, EMBED_DIM, BATCH),
                                      jnp.float32),
        compiler_params=pltpu.CompilerParams(needs_layout_passes=False),
        scratch_types=[
            pltpu.VMEM((_B_PER_W + _L,), jnp.int32),
            pltpu.VMEM((_BC * _NFMAX, 1, EMBED_DIM), jnp.float32),
            pltpu.VMEM((_BC * _NFMAX, 1, EMBED_DIM), jnp.float32),
            pltpu.VMEM((_NFMAX, EMBED_DIM, _BPW), jnp.float32),
            pltpu.SemaphoreType.DMA,
            pltpu.SemaphoreType.DMA,
            pltpu.SemaphoreType.DMA,
        ],
    )
    def gather_kernel(table_hbm, idx_hbm, out_hbm, idx_v,
                      rb0, rb1, tbuf, g0, g1, osem):
        wid = lax.axis_index("s") * _NC + lax.axis_index("c")
        base = pl.multiple_of(wid * _B_PER_W, _B_PER_W)
        b_base = pl.multiple_of(wid * _BPW, _BPW)
        pltpu.sync_copy(idx_hbm.at[pl.ds(base, _B_PER_W)],
                        idx_v.at[pl.ds(0, _B_PER_W)])

        rbufs = (rb0, rb1)
        gsems = (g0, g1)
        iota16 = lax.iota(jnp.int32, _L)
        zv = iota16 * 0

        def fill(f0, nf, bc, parity):
            rbuf = rbufs[parity]
            sem = gsems[parity]

            def body(br, carry):
                v = idx_v[pl.ds((bc * _BC + br) * FIELDS + f0, _L)]
                for df in range(nf):
                    i = v[df]
                    pltpu.async_copy(
                        table_hbm.at[pl.ds(i >> 3, 1), pl.ds(i & 7, 1),
                                     pl.ds(0, EMBED_DIM)],
                        rbuf.at[pl.ds(br * nf + df, 1)],
                        sem,
                    )
                return carry

            lax.fori_loop(0, _BC, body, 0)

        def drain_fill(nf, parity):
            pltpu.make_async_copy(
                table_hbm.at[pl.ds(0, _BC * nf), pl.ds(0, 1),
                             pl.ds(0, EMBED_DIM)],
                rbufs[parity].at[pl.ds(0, _BC * nf)], gsems[parity]
            ).wait()

        def transpose(nf, bc, parity):
            rbuf = rbufs[parity]
            for df in range(nf):
                r_idx = iota16 * nf + df

                def kbody(kq, carry):
                    k0 = kq * 4
                    for dk in range(4):
                        tbuf[df, k0 + dk, pl.ds(bc * _BC, _BC)] = (
                            plsc.load_gather(rbuf, [r_idx, zv, zv + (k0 + dk)])
                        )
                    return carry

                lax.fori_loop(0, EMBED_DIM // 4, kbody, 0)

        def put(f0, nf):
            return pltpu.async_copy(
                tbuf.at[pl.ds(0, nf)],
                out_hbm.at[pl.ds(f0, nf), pl.ds(0, EMBED_DIM),
                           pl.ds(b_base, _BPW)],
                osem,
            )

        for p, (f0, nf) in enumerate(_GROUPS):
            if p > 0:
                pltpu.make_async_copy(
                    out_hbm.at[pl.ds(0, _GROUPS[p - 1][1]),
                               pl.ds(0, EMBED_DIM), pl.ds(0, _BPW)],
                    tbuf.at[pl.ds(0, _GROUPS[p - 1][1])], osem
                ).wait()
            fill(f0, nf, 0, 0)

            def pair_body(q, carry):
                bc0 = q * 2
                drain_fill(nf, 0)
                fill(f0, nf, bc0 + 1, 1)
                transpose(nf, bc0, 0)
                drain_fill(nf, 1)

                @pl.when(q < _NCH // 2 - 1)
                def _():
                    fill(f0, nf, bc0 + 2, 0)

                transpose(nf, bc0 + 1, 1)
                return carry

            lax.fori_loop(0, _NCH // 2, pair_body, 0)
            put(f0, nf)
        pltpu.make_async_copy(
            out_hbm.at[pl.ds(0, _GROUPS[-1][1]), pl.ds(0, EMBED_DIM),
                       pl.ds(0, _BPW)],
            tbuf.at[pl.ds(0, _GROUPS[-1][1])], osem
        ).wait()

    return gather_kernel


_gather = _make_kernel()


def kernel(inp, weight):
    idx = inp.reshape(-1).astype(jnp.int32)
    table_tiles = weight.reshape(_VT, 8, EMBED_DIM)
    outt = _gather(table_tiles, idx)
    return outt.transpose(2, 0, 1)


# speedup vs baseline: 1.2615x; 1.2534x over previous
import functools

import jax
import jax.numpy as jnp
from jax import lax
from jax.experimental import pallas as pl
from jax.experimental.pallas import tpu as pltpu
from jax.experimental.pallas import tpu_sc as plsc

VOCAB = 1000000
EMBED_DIM = 64
BATCH = 4096
FIELDS = 26

_B = BATCH * FIELDS
_VT = VOCAB // 8

_info = plsc.get_sparse_core_info()
_NC, _NS = _info.num_cores, _info.num_subcores
_NW = _NC * _NS
_B_PER_W = _B // _NW
_CH = 208
_NP = _CH // 2
_N_CHUNKS = _B_PER_W // _CH
_L = 16


def _make_kernel():
    mesh = plsc.VectorSubcoreMesh(core_axis_name="c", subcore_axis_name="s")

    @functools.partial(
        pl.kernel,
        mesh=mesh,
        out_type=jax.ShapeDtypeStruct((BATCH, FIELDS, EMBED_DIM),
                                      jnp.float32),
        scratch_types=[
            pltpu.VMEM((_B_PER_W,), jnp.int32),
            pltpu.VMEM((_NP, 2, EMBED_DIM), jnp.float32),
            pltpu.VMEM((_NP, 2, EMBED_DIM), jnp.float32),
            pltpu.SemaphoreType.DMA,
            pltpu.SemaphoreType.DMA,
            pltpu.SemaphoreType.DMA,
            pltpu.SemaphoreType.DMA,
        ],
    )
    def gather_kernel(table_hbm, idx_hbm, out_hbm, idx_v,
                      rows0, rows1, g0, g1, o0, o1):
        wid = lax.axis_index("s") * _NC + lax.axis_index("c")
        base = pl.multiple_of(wid * _B_PER_W, _B_PER_W)
        b_base = pl.multiple_of(wid * (BATCH // _NW), BATCH // _NW)
        pltpu.sync_copy(idx_hbm.at[pl.ds(base, _B_PER_W)], idx_v)

        bufs = (rows0, rows1)
        gsems = (g0, g1)
        osems = (o0, o1)

        def fill(c):
            buf = bufs[c % 2]
            sem = gsems[c % 2]

            def body(g, carry):
                v = idx_v[pl.ds(c * _CH + g * _L, _L)]
                for t in range(_L):
                    i = v[t]
                    pltpu.async_copy(
                        table_hbm.at[pl.ds(i >> 3, 1), pl.ds(i & 7, 1),
                                     pl.ds(0, EMBED_DIM)],
                        buf.at[pl.ds(g * (_L // 2) + t // 2, 1),
                               pl.ds(t & 1, 1), pl.ds(0, EMBED_DIM)],
                        sem,
                    )
                return carry

            lax.fori_loop(0, _CH // _L, body, 0)

        def drain(c, sems):
            pltpu.make_async_copy(
                table_hbm.at[pl.ds(0, _NP), pl.ds(0, 2), pl.ds(0, EMBED_DIM)],
                bufs[c % 2], sems[c % 2]
            ).wait()

        def put_pairs(c):
            buf = bufs[c % 2]
            sem = osems[c % 2]
            row0 = c * _CH

            def body(g, carry):
                for t in range(8):
                    q = g * 8 + t
                    rj = row0 + 2 * q
                    b = b_base + rj // FIELDS
                    f0 = rj % FIELDS
                    pltpu.async_copy(
                        buf.at[pl.ds(q, 1)],
                        out_hbm.at[pl.ds(b, 1), pl.ds(f0, 2),
                                   pl.ds(0, EMBED_DIM)],
                        sem,
                    )
                return carry

            lax.fori_loop(0, _NP // 8, body, 0)

        for c in range(_N_CHUNKS):
            if c >= 2:
                drain(c, osems)
            fill(c)
            drain(c, gsems)
            put_pairs(c)
        drain(_N_CHUNKS - 2, osems)
        drain(_N_CHUNKS - 1, osems)

    return gather_kernel


_gather = _make_kernel()


def kernel(inp, weight):
    idx = inp.reshape(-1).astype(jnp.int32)
    table_tiles = weight.reshape(_VT, 8, EMBED_DIM)
    return _gather(table_tiles, idx)
